# baseline (speedup 1.0000x reference)
import jax
import jax.numpy as jnp
from jax import lax
from jax.experimental import pallas as pl
from jax.experimental.pallas import tpu as pltpu
from jax.experimental.pallas import tpu_sc as plsc

_VOCABS = (6, 7, 12, 7, 96)
_B = 16384
_XD = 64
_D = 10
_DP = 16
_NT = 5
_OW = _XD + _NT * _D
_CV = sum(_VOCABS)

_NC = 2
_NS = 16
_NW = _NC * _NS
_BPW = _B // _NW
_NG = _BPW // 16
_UNROLL = 2

_TBASE = []
_acc = 0
for _v in _VOCABS:
    _TBASE.append(_acc * _DP)
    _acc += _v


def _body(xt_hbm, yt_hbm, tcat_hbm, ot_hbm, yv, tv, otv, xsem, sem):
    wid = lax.axis_index("s") * _NC + lax.axis_index("c")
    b0 = wid * _BPW

    xcp = pltpu.make_async_copy(
        xt_hbm.at[:, pl.ds(b0, _BPW)], otv.at[pl.ds(0, _XD), :], xsem
    )
    xcp.start()
    pltpu.sync_copy(tcat_hbm, tv)
    pltpu.sync_copy(yt_hbm.at[:, pl.ds(b0, _BPW)], yv)

    def assemble(it, _):
        for u in range(_UNROLL):
            p = it * _UNROLL + u
            for i in range(_NT):
                y16 = yv[i, pl.ds(16 * p, 16)]
                base = y16 * _DP + _TBASE[i]
                for j in range(_D):
                    col = plsc.load_gather(tv, [base + j])
                    otv[_XD + _D * i + j, pl.ds(16 * p, 16)] = col
        return ()

    lax.fori_loop(0, _NG // _UNROLL, assemble, (), unroll=2)

    xcp.wait()
    pltpu.sync_copy(otv, ot_hbm.at[:, pl.ds(b0, _BPW)])


_sc_call = pl.kernel(
    _body,
    out_type=jax.ShapeDtypeStruct((_OW, _B), jnp.float32),
    mesh=plsc.VectorSubcoreMesh(core_axis_name="c", subcore_axis_name="s"),
    scratch_types=[
        pltpu.VMEM((_NT, _BPW), jnp.int32),
        pltpu.VMEM((_CV * _DP,), jnp.float32),
        pltpu.VMEM((_OW, _BPW), jnp.float32),
        pltpu.SemaphoreType.DMA,
        pltpu.SemaphoreType.DMA,
    ],
    compiler_params=pltpu.CompilerParams(
        use_tc_tiling_on_sc=True,
        needs_layout_passes=False,
        skip_device_barrier=True,
    ),
)


def kernel(x, y, emb0, emb1, emb2, emb3, emb4):
    tcat = jnp.concatenate([emb0, emb1, emb2, emb3, emb4], axis=0)
    tcat = jnp.pad(tcat, ((0, 0), (0, _DP - _D))).reshape(-1)
    out_t = _sc_call(x.T, y.astype(jnp.int32).T, tcat)
    return out_t.T


# speedup vs baseline: 1.0154x; 1.0154x over previous
import jax
import jax.numpy as jnp
from jax import lax
from jax.experimental import pallas as pl
from jax.experimental.pallas import tpu as pltpu
from jax.experimental.pallas import tpu_sc as plsc

_VOCABS = (6, 7, 12, 7, 96)
_B = 16384
_XD = 64
_D = 10
_DP = 16
_NT = 5
_OW = _XD + _NT * _D
_CV = sum(_VOCABS)

_NC = 2
_NS = 16
_NW = _NC * _NS
_BPW = _B // _NW
_NG = _BPW // 16
_UNROLL = 2

_TBASE = []
_acc = 0
for _v in _VOCABS:
    _TBASE.append(_acc * _DP)
    _acc += _v


_HALF = _BPW // 2
_HG = _NG // 2


def _body(xt_hbm, yt_hbm, tcat_hbm, ot_hbm, yv, tv, otv, xs0, xs1, osem):
    wid = lax.axis_index("s") * _NC + lax.axis_index("c")
    b0 = wid * _BPW

    xcps = []
    for h, xs in ((0, xs0), (1, xs1)):
        c = pltpu.make_async_copy(
            xt_hbm.at[:, pl.ds(b0 + h * _HALF, _HALF)],
            otv.at[pl.ds(0, _XD), pl.ds(h * _HALF, _HALF)],
            xs,
        )
        c.start()
        xcps.append(c)
    pltpu.sync_copy(tcat_hbm, tv)
    pltpu.sync_copy(yt_hbm.at[:, pl.ds(b0, _BPW)], yv)

    def make_assemble(h):
        def assemble(it, _):
            for u in range(_UNROLL):
                p = h * _HG + it * _UNROLL + u
                for i in range(_NT):
                    y16 = yv[i, pl.ds(16 * p, 16)]
                    base = y16 * _DP + _TBASE[i]
                    for j in range(_D):
                        col = plsc.load_gather(tv, [base + j])
                        otv[_XD + _D * i + j, pl.ds(16 * p, 16)] = col
            return ()
        return assemble

    ocps = []
    for h in range(2):
        lax.fori_loop(0, _HG // _UNROLL, make_assemble(h), (), unroll=2)
        xcps[h].wait()
        oc = pltpu.make_async_copy(
            otv.at[:, pl.ds(h * _HALF, _HALF)],
            ot_hbm.at[:, pl.ds(b0 + h * _HALF, _HALF)],
            osem,
        )
        oc.start()
        ocps.append(oc)
    for oc in ocps:
        oc.wait()


_sc_call = pl.kernel(
    _body,
    out_type=jax.ShapeDtypeStruct((_OW, _B), jnp.float32),
    mesh=plsc.VectorSubcoreMesh(core_axis_name="c", subcore_axis_name="s"),
    scratch_types=[
        pltpu.VMEM((_NT, _BPW), jnp.int32),
        pltpu.VMEM((_CV * _DP,), jnp.float32),
        pltpu.VMEM((_OW, _BPW), jnp.float32),
        pltpu.SemaphoreType.DMA,
        pltpu.SemaphoreType.DMA,
        pltpu.SemaphoreType.DMA,
    ],
    compiler_params=pltpu.CompilerParams(
        use_tc_tiling_on_sc=True,
        needs_layout_passes=False,
    ),
)


def kernel(x, y, emb0, emb1, emb2, emb3, emb4):
    tcat = jnp.concatenate([emb0, emb1, emb2, emb3, emb4], axis=0)
    tcat = jnp.pad(tcat, ((0, 0), (0, _DP - _D))).reshape(-1)
    out_t = _sc_call(x.T, y.astype(jnp.int32).T, tcat)
    return out_t.T
